# baseline (speedup 1.0000x reference)
import functools

import jax
import jax.numpy as jnp
from jax import lax
from jax.experimental import pallas as pl
from jax.experimental.pallas import tpu as pltpu
from jax.experimental.pallas import tpu_sc as plsc


def _argmin_kernel(z_ref, cbm2_ref, c2_ref, idx_ref, vq_ref):
    zb = z_ref[0]
    cbm2 = cbm2_ref[0]
    c2 = c2_ref[0]

    xcm2 = jax.lax.dot_general(
        cbm2, zb, (((1,), (0,)), ((), ())),
        preferred_element_type=jnp.float32)
    x2 = jnp.sum(zb * zb, axis=0, keepdims=True)
    scores = (x2 + c2) + xcm2

    idx_ref[0, 0, 0] = jnp.argmin(scores, axis=0)
    mins = jnp.min(scores, axis=0)
    vq_ref[...] = jnp.sum(mins).reshape(1, 1, 1, 1)


def _make_sc_gather(N, B, ds, M, HW):
    mesh = plsc.VectorSubcoreMesh(core_axis_name="c", subcore_axis_name="s")
    info = plsc.get_sparse_core_info()
    NC = info.num_cores
    L = info.num_lanes
    chunks = HW // L

    @functools.partial(
        pl.kernel,
        mesh=mesh,
        out_type=jax.ShapeDtypeStruct((B, N * ds, HW), jnp.float32),
        scratch_types=[
            pltpu.VMEM((ds * M,), jnp.float32),
            pltpu.VMEM((HW,), jnp.int32),
            pltpu.VMEM((ds, HW), jnp.float32),
        ],
        compiler_params=pltpu.CompilerParams(needs_layout_passes=False),
    )
    def sc_gather(cbt_hbm, idx_hbm, out_hbm, cbt_v, idx_v, q_v):
        n = lax.axis_index("s") * NC + lax.axis_index("c")
        pltpu.sync_copy(cbt_hbm.at[n], cbt_v)

        def b_body(b, carry):
            pltpu.sync_copy(idx_hbm.at[b, n], idx_v)

            def c_body(c, carry2):
                t0 = c * L
                iv = idx_v[pl.ds(t0, L)]
                for d in range(ds):
                    g = plsc.load_gather(cbt_v, [iv + (d * M)])
                    q_v[d, pl.ds(t0, L)] = g
                return carry2

            lax.fori_loop(0, chunks, c_body, 0, unroll=2)
            pltpu.sync_copy(q_v, out_hbm.at[b, pl.ds(n * ds, ds), :])
            return carry

        lax.fori_loop(0, B, b_body, 0)

    return sc_gather


@functools.partial(jax.jit, static_argnames=())
def kernel(z, cb):
    beta = 0.25
    B, D, H, W = z.shape
    N, M, ds = cb.shape
    T = B * H * W
    HW = H * W

    zr = z.reshape(B, D, HW)
    c2 = jnp.sum(cb * cb, axis=2, keepdims=True)
    cb_m2 = -2.0 * cb
    cbt = jnp.transpose(cb, (0, 2, 1)).reshape(N, ds * M)

    idx, vq = pl.pallas_call(
        _argmin_kernel,
        grid=(N, B),
        in_specs=[
            pl.BlockSpec((1, ds, HW), lambda n, b: (b, n, 0)),
            pl.BlockSpec((1, M, ds), lambda n, b: (n, 0, 0)),
            pl.BlockSpec((1, M, 1), lambda n, b: (n, 0, 0)),
        ],
        out_specs=[
            pl.BlockSpec((1, 1, 1, HW), lambda n, b: (b, n, 0, 0)),
            pl.BlockSpec((1, 1, 1, 1), lambda n, b: (n, b, 0, 0)),
        ],
        out_shape=[
            jax.ShapeDtypeStruct((B, N, 1, HW), jnp.int32),
            jax.ShapeDtypeStruct((N, B, 1, 1), jnp.float32),
        ],
        compiler_params=pltpu.CompilerParams(
            dimension_semantics=("parallel", "parallel")),
    )(zr, cb_m2, c2)

    zq = _make_sc_gather(N, B, ds, M, HW)(cbt, idx.reshape(B, N, HW))

    z_q = zq.reshape(B, D, H, W)
    indices = idx.reshape(B, N, H, W)
    vq_loss = (1.0 + beta) * jnp.sum(vq) / (N * T * ds)
    return (z_q, vq_loss, indices)


# speedup vs baseline: 1.3737x; 1.3737x over previous
import functools

import jax
import jax.numpy as jnp
from jax import lax
from jax.experimental import pallas as pl
from jax.experimental.pallas import tpu as pltpu
from jax.experimental.pallas import tpu_sc as plsc


def _argmin_kernel(z_ref, cbm2_ref, c2_ref, idx_ref, vq_ref):
    N = cbm2_ref.shape[0]
    ds = cbm2_ref.shape[2]
    acc = jnp.zeros((), jnp.float32)
    for n in range(N):
        zb = z_ref[0, n * ds:(n + 1) * ds]
        cbm2 = cbm2_ref[n]
        c2 = c2_ref[n]

        xcm2 = jax.lax.dot_general(
            cbm2, zb, (((1,), (0,)), ((), ())),
            preferred_element_type=jnp.float32)
        x2 = jnp.sum(zb * zb, axis=0, keepdims=True)
        scores = (x2 + c2) + xcm2

        idx_ref[0, n, 0] = jnp.argmin(scores, axis=0)
        acc += jnp.sum(jnp.min(scores, axis=0))
    vq_ref[...] = acc.reshape(1, 1, 1, 1)


def _make_sc_gather(N, B, ds, M, HW):
    mesh = plsc.VectorSubcoreMesh(core_axis_name="c", subcore_axis_name="s")
    info = plsc.get_sparse_core_info()
    NC = info.num_cores
    L = info.num_lanes
    chunks = HW // L

    @functools.partial(
        pl.kernel,
        mesh=mesh,
        out_type=jax.ShapeDtypeStruct((B, N * ds, HW), jnp.float32),
        scratch_types=[
            pltpu.VMEM((ds * M,), jnp.float32),
            pltpu.VMEM((HW,), jnp.int32),
            pltpu.VMEM((ds, HW), jnp.float32),
        ],
        compiler_params=pltpu.CompilerParams(needs_layout_passes=False),
    )
    def sc_gather(cbt_hbm, idx_hbm, out_hbm, cbt_v, idx_v, q_v):
        n = lax.axis_index("s") * NC + lax.axis_index("c")
        pltpu.sync_copy(cbt_hbm.at[n], cbt_v)

        def b_body(b, carry):
            pltpu.sync_copy(idx_hbm.at[b, n], idx_v)

            def c_body(c, carry2):
                t0 = c * L
                iv = idx_v[pl.ds(t0, L)]
                for d in range(ds):
                    g = plsc.load_gather(cbt_v, [iv + (d * M)])
                    q_v[d, pl.ds(t0, L)] = g
                return carry2

            lax.fori_loop(0, chunks, c_body, 0, unroll=2)
            pltpu.sync_copy(q_v, out_hbm.at[b, pl.ds(n * ds, ds), :])
            return carry

        lax.fori_loop(0, B, b_body, 0)

    return sc_gather


@functools.partial(jax.jit, static_argnames=())
def kernel(z, cb):
    beta = 0.25
    B, D, H, W = z.shape
    N, M, ds = cb.shape
    T = B * H * W
    HW = H * W

    zr = z.reshape(B, D, HW)
    c2 = jnp.sum(cb * cb, axis=2, keepdims=True)
    cb_m2 = -2.0 * cb
    cbt = jnp.transpose(cb, (0, 2, 1)).reshape(N, ds * M)

    idx, vq = pl.pallas_call(
        _argmin_kernel,
        grid=(B,),
        in_specs=[
            pl.BlockSpec((1, D, HW), lambda b: (b, 0, 0)),
            pl.BlockSpec((N, M, ds), lambda b: (0, 0, 0)),
            pl.BlockSpec((N, M, 1), lambda b: (0, 0, 0)),
        ],
        out_specs=[
            pl.BlockSpec((1, N, 1, HW), lambda b: (b, 0, 0, 0)),
            pl.BlockSpec((1, 1, 1, 1), lambda b: (b, 0, 0, 0)),
        ],
        out_shape=[
            jax.ShapeDtypeStruct((B, N, 1, HW), jnp.int32),
            jax.ShapeDtypeStruct((B, 1, 1, 1), jnp.float32),
        ],
        compiler_params=pltpu.CompilerParams(
            dimension_semantics=("parallel",)),
    )(zr, cb_m2, c2)

    zq = _make_sc_gather(N, B, ds, M, HW)(cbt, idx.reshape(B, N, HW))

    z_q = zq.reshape(B, D, H, W)
    indices = idx.reshape(B, N, H, W)
    vq_loss = (1.0 + beta) * jnp.sum(vq) / (N * T * ds)
    return (z_q, vq_loss, indices)


# speedup vs baseline: 1.4411x; 1.0491x over previous
import functools

import jax
import jax.numpy as jnp
from jax import lax
from jax.experimental import pallas as pl
from jax.experimental.pallas import tpu as pltpu
from jax.experimental.pallas import tpu_sc as plsc


def _argmin_kernel(z_ref, cbm2_ref, c2_ref, idx_ref):
    N = cbm2_ref.shape[0]
    ds = cbm2_ref.shape[2]
    for n in range(N):
        zb = z_ref[0, n * ds:(n + 1) * ds]
        cbm2 = cbm2_ref[n]
        c2 = c2_ref[n]

        xcm2 = jax.lax.dot_general(
            cbm2, zb, (((1,), (0,)), ((), ())),
            preferred_element_type=jnp.float32)
        x2 = jnp.sum(zb * zb, axis=0, keepdims=True)
        scores = (x2 + c2) + xcm2

        idx_ref[0, n, 0] = jnp.argmin(scores, axis=0)


def _make_sc_gather(N, B, ds, M, HW):
    mesh = plsc.VectorSubcoreMesh(core_axis_name="c", subcore_axis_name="s")
    info = plsc.get_sparse_core_info()
    NC = info.num_cores
    L = info.num_lanes
    chunks = HW // L

    NW = NC * info.num_subcores

    @functools.partial(
        pl.kernel,
        mesh=mesh,
        out_type=[
            jax.ShapeDtypeStruct((B, N * ds, HW), jnp.float32),
            jax.ShapeDtypeStruct((NW, L), jnp.float32),
        ],
        scratch_types=[
            pltpu.VMEM((ds * M,), jnp.float32),
            pltpu.VMEM((HW,), jnp.int32),
            pltpu.VMEM((ds, HW), jnp.float32),
            pltpu.VMEM((ds, HW), jnp.float32),
            pltpu.VMEM((L,), jnp.float32),
        ],
        compiler_params=pltpu.CompilerParams(needs_layout_passes=False),
    )
    def sc_gather(cbt_hbm, idx_hbm, z_hbm, out_hbm, vqp_hbm,
                  cbt_v, idx_v, q_v, z_v, acc_v):
        n = lax.axis_index("s") * NC + lax.axis_index("c")
        pltpu.sync_copy(cbt_hbm.at[n], cbt_v)

        def b_body(b, acc):
            pltpu.sync_copy(idx_hbm.at[b, n], idx_v)
            pltpu.sync_copy(z_hbm.at[b, pl.ds(n * ds, ds), :], z_v)

            def c_body(c, acc2):
                t0 = c * L
                iv = idx_v[pl.ds(t0, L)]
                for d in range(ds):
                    g = plsc.load_gather(cbt_v, [iv + (d * M)])
                    q_v[d, pl.ds(t0, L)] = g
                    df = z_v[d, pl.ds(t0, L)] - g
                    acc2 = acc2 + df * df
                return acc2

            acc = lax.fori_loop(0, chunks, c_body, acc, unroll=2)
            pltpu.sync_copy(q_v, out_hbm.at[b, pl.ds(n * ds, ds), :])
            return acc

        acc = lax.fori_loop(0, B, b_body, jnp.zeros((L,), jnp.float32))
        acc_v[...] = acc
        pltpu.sync_copy(acc_v, vqp_hbm.at[n])

    return sc_gather


@functools.partial(jax.jit, static_argnames=())
def kernel(z, cb):
    beta = 0.25
    B, D, H, W = z.shape
    N, M, ds = cb.shape
    T = B * H * W
    HW = H * W

    zr = z.reshape(B, D, HW)
    c2 = jnp.sum(cb * cb, axis=2, keepdims=True)
    cb_m2 = -2.0 * cb
    cbt = jnp.transpose(cb, (0, 2, 1)).reshape(N, ds * M)

    idx = pl.pallas_call(
        _argmin_kernel,
        grid=(B,),
        in_specs=[
            pl.BlockSpec((1, D, HW), lambda b: (b, 0, 0)),
            pl.BlockSpec((N, M, ds), lambda b: (0, 0, 0)),
            pl.BlockSpec((N, M, 1), lambda b: (0, 0, 0)),
        ],
        out_specs=pl.BlockSpec((1, N, 1, HW), lambda b: (b, 0, 0, 0)),
        out_shape=jax.ShapeDtypeStruct((B, N, 1, HW), jnp.int32),
        compiler_params=pltpu.CompilerParams(
            dimension_semantics=("parallel",)),
    )(zr, cb_m2, c2)

    zq, vqp = _make_sc_gather(N, B, ds, M, HW)(
        cbt, idx.reshape(B, N, HW), zr)

    z_q = zq.reshape(B, D, H, W)
    indices = idx.reshape(B, N, H, W)
    vq_loss = (1.0 + beta) * jnp.sum(vqp) / (N * T * ds)
    return (z_q, vq_loss, indices)
